# Initial kernel scaffold; baseline (speedup 1.0000x reference)
import jax
import jax.numpy as jnp
from jax.experimental import pallas as pl


def kernel(node_embedding, edge_embedding, static_edge_features, static_graph_features, static_rt_graph_features, edge_index, graph_ids, W_root_0, W_nbr_0, b_0, W_root_1, W_nbr_1, b_1, W_root_2, W_nbr_2, b_2, We_o, be_o, Wp_o, bp_o, Wr_o, br_o, Wc_o, bc_o):
    raise NotImplementedError("write your pallas kernel here")


# speedup vs baseline: 1.7763x; 1.7763x over previous
import functools

import jax
import jax.numpy as jnp
from jax import lax
from jax.experimental import pallas as pl
from jax.experimental.pallas import tpu as pltpu
from jax.experimental.pallas import tpu_sc as plsc

NC = 2
NS = 16
NW = NC * NS
CHUNK = 125


NPH = 3
AROWS = 3712


def _make_sc_agg(n, d, e):
    arows = AROWS
    urows = arows - 128
    npad = NPH * urows
    zpt = arows // NS
    upt = urows // NS
    ept = e // NS
    nit = ept // CHUNK
    mesh = plsc.VectorSubcoreMesh(core_axis_name="c", subcore_axis_name="s",
                                  num_cores=1)

    @functools.partial(
        pl.kernel,
        mesh=mesh,
        out_type=[
            jax.ShapeDtypeStruct((npad, d), jnp.float32),
            jax.ShapeDtypeStruct((npad,), jnp.float32),
        ],
        scratch_types=[
            pltpu.VMEM((nit, CHUNK), jnp.int32),
            pltpu.VMEM((nit, CHUNK), jnp.int32),
            pltpu.VMEM((CHUNK, d), jnp.float32),
            pltpu.VMEM((CHUNK,), jnp.float32),
            pltpu.VMEM((zpt, d), jnp.float32),
            pltpu.VMEM((zpt,), jnp.float32),
            pltpu.VMEM_SHARED((arows, d), jnp.float32),
            pltpu.VMEM_SHARED((arows,), jnp.float32),
            pltpu.SemaphoreType.DMA,
        ],
    )
    def agg(x_h, src_h, dst3_h, z2_h, z1_h, one_h, outp, outd,
            sidx, didx, rows, onesv, zbuf, zd, acc, dacc, sem):
        s = lax.axis_index("s")
        pltpu.sync_copy(src_h.at[pl.ds(s * nit, nit)], sidx)
        pltpu.sync_copy(one_h, onesv)
        pltpu.sync_copy(z2_h, zbuf)
        pltpu.sync_copy(z1_h, zd)
        for p in range(NPH):
            pltpu.sync_copy(dst3_h.at[p, pl.ds(s * nit, nit)], didx)
            pltpu.sync_copy(zbuf, acc.at[pl.ds(s * zpt, zpt)])
            pltpu.sync_copy(zd, dacc.at[pl.ds(s * zpt, zpt)])
            plsc.subcore_barrier()

            def step(j, carry):
                pltpu.async_copy(x_h.at[sidx.at[j]], rows, sem).wait()
                pltpu.sync_copy(rows, acc.at[didx.at[j]], add=True)
                pltpu.sync_copy(onesv, dacc.at[didx.at[j]], add=True)
                return carry

            lax.fori_loop(0, nit, step, 0)
            plsc.subcore_barrier()
            pltpu.sync_copy(acc.at[pl.ds(s * upt, upt)],
                            zbuf.at[pl.ds(0, upt)])
            pltpu.sync_copy(zbuf.at[pl.ds(0, upt)],
                            outp.at[pl.ds(p * urows + s * upt, upt)])
            pltpu.sync_copy(dacc.at[pl.ds(s * upt, upt)], zd.at[pl.ds(0, upt)])
            pltpu.sync_copy(zd.at[pl.ds(0, upt)],
                            outd.at[pl.ds(p * urows + s * upt, upt)])
            plsc.subcore_barrier()
            pltpu.sync_copy(z2_h, zbuf)
            pltpu.sync_copy(z1_h, zd)

    return agg, npad, urows


def _make_sc_edge(n, e):
    epw = e // NW
    nit = epw // 16
    mesh = plsc.VectorSubcoreMesh(core_axis_name="c", subcore_axis_name="s")

    @functools.partial(
        pl.kernel,
        mesh=mesh,
        out_type=jax.ShapeDtypeStruct((2 * e,), jnp.float32),
        compiler_params=pltpu.CompilerParams(needs_layout_passes=False),
        scratch_types=[
            pltpu.VMEM((4 * n,), jnp.float32),
            pltpu.VMEM((epw,), jnp.int32),
            pltpu.VMEM((epw,), jnp.int32),
            pltpu.VMEM((2 * epw,), jnp.float32),
            pltpu.VMEM((2 * epw,), jnp.float32),
        ],
    )
    def edge(y_h, src_h, dst_h, contrib_h, out_h, ytab, sidx, didx, cv, obuf):
        c = lax.axis_index("c")
        s = lax.axis_index("s")
        wid = c * NS + s
        base = wid * epw
        pltpu.sync_copy(y_h, ytab)
        pltpu.sync_copy(src_h.at[pl.ds(base, epw)], sidx)
        pltpu.sync_copy(dst_h.at[pl.ds(base, epw)], didx)
        pltpu.sync_copy(contrib_h.at[pl.ds(2 * base, 2 * epw)], cv)
        iota = lax.broadcasted_iota(jnp.int32, (16,), 0)

        def step(i, carry):
            s16 = sidx[pl.ds(i * 16, 16)]
            d16 = didx[pl.ds(i * 16, 16)]
            r16 = i * 32 + 2 * iota
            v0 = plsc.load_gather(ytab, [s16 * 4])
            v1 = plsc.load_gather(ytab, [s16 * 4 + 1])
            v2 = plsc.load_gather(ytab, [d16 * 4 + 2])
            v3 = plsc.load_gather(ytab, [d16 * 4 + 3])
            g0 = plsc.load_gather(cv, [r16])
            g1 = plsc.load_gather(cv, [r16 + 1])
            plsc.store_scatter(obuf, [r16], v0 + v2 + g0)
            plsc.store_scatter(obuf, [r16 + 1], v1 + v3 + g1)
            return carry

        lax.fori_loop(0, nit, step, 0)
        pltpu.sync_copy(obuf, out_h.at[pl.ds(2 * base, 2 * epw)])

    return edge


def _tc_layer_body(x_ref, p_ref, dt_ref, wr_ref, wn_ref, b_ref, wsd_ref,
                   o_ref, y_ref):
    inv = 1.0 / jnp.maximum(dt_ref[...], 1.0)
    agg = p_ref[...] * inv
    h = (jnp.dot(x_ref[...], wr_ref[...], preferred_element_type=jnp.float32)
         + jnp.dot(agg, wn_ref[...], preferred_element_type=jnp.float32)
         + b_ref[...])
    xo = jnp.where(h > 0, h, jnp.exp(h) - 1.0)
    o_ref[...] = xo
    y_ref[...] = jnp.dot(xo, wsd_ref[...], preferred_element_type=jnp.float32)


def _tc_layer(x, p, degc, wr, wn, b, wsd, bn=2000):
    n, d = x.shape
    grid = (n // bn,)
    return pl.pallas_call(
        _tc_layer_body,
        grid=grid,
        in_specs=[
            pl.BlockSpec((bn, d), lambda i: (i, 0)),
            pl.BlockSpec((bn, d), lambda i: (i, 0)),
            pl.BlockSpec((bn, 1), lambda i: (i, 0)),
            pl.BlockSpec((d, d), lambda i: (0, 0)),
            pl.BlockSpec((d, d), lambda i: (0, 0)),
            pl.BlockSpec((1, d), lambda i: (0, 0)),
            pl.BlockSpec((d, 4), lambda i: (0, 0)),
        ],
        out_specs=[
            pl.BlockSpec((bn, d), lambda i: (i, 0)),
            pl.BlockSpec((bn, 4), lambda i: (i, 0)),
        ],
        out_shape=[
            jax.ShapeDtypeStruct((n, d), jnp.float32),
            jax.ShapeDtypeStruct((n, 4), jnp.float32),
        ],
    )(x, p, degc, wr, wn, b, wsd)


def _tc_contrib_body(ee_ref, se_ref, wee_ref, wse_ref, b_ref, o_ref):
    o_ref[...] = (
        jnp.dot(ee_ref[...], wee_ref[...], preferred_element_type=jnp.float32)
        + jnp.dot(se_ref[...], wse_ref[...], preferred_element_type=jnp.float32)
        + b_ref[...])


def _tc_contrib(ee, se, wee, wse, b2, be=1280):
    e, d = ee.shape
    sf = se.shape[1]
    return pl.pallas_call(
        _tc_contrib_body,
        grid=(e // be,),
        in_specs=[
            pl.BlockSpec((be, d), lambda i: (i, 0)),
            pl.BlockSpec((be, sf), lambda i: (i, 0)),
            pl.BlockSpec((d, 2), lambda i: (0, 0)),
            pl.BlockSpec((sf, 2), lambda i: (0, 0)),
            pl.BlockSpec((1, 2), lambda i: (0, 0)),
        ],
        out_specs=pl.BlockSpec((be, 2), lambda i: (i, 0)),
        out_shape=jax.ShapeDtypeStruct((e, 2), jnp.float32),
    )(ee, se, wee, wse, b2)


def _make_tc_pool(n, g):
    def body(x_ref, gid_ref, sgf_ref, srt_ref, wpx_ref, wps_ref, wrx_ref,
             wrs_ref, wcx_ref, wcs_ref, bp_ref, br_ref, bc_ref,
             prec_ref, rt_ref, ccs_ref):
        gid = gid_ref[...]
        oh = (lax.broadcasted_iota(jnp.int32, (g, n), 0) == gid)
        ohf = oh.astype(jnp.float32)
        accp = jnp.dot(ohf, x_ref[...], preferred_element_type=jnp.float32)
        accn = jnp.sum(ohf, axis=1, keepdims=True)
        pooled = accp / jnp.maximum(accn, 1.0)
        prec_ref[...] = (
            jnp.dot(pooled, wpx_ref[...], preferred_element_type=jnp.float32)
            + jnp.dot(sgf_ref[...], wps_ref[...], preferred_element_type=jnp.float32)
            + bp_ref[...])
        rt_ref[...] = (
            jnp.dot(pooled, wrx_ref[...], preferred_element_type=jnp.float32)
            + jnp.dot(srt_ref[...], wrs_ref[...], preferred_element_type=jnp.float32)
            + br_ref[...])
        ccs_ref[...] = (
            jnp.dot(pooled, wcx_ref[...], preferred_element_type=jnp.float32)
            + jnp.dot(srt_ref[...], wcs_ref[...], preferred_element_type=jnp.float32)
            + bc_ref[...])

    return body


def _tc_pool(x, gidf, sgf, srt, wpx, wps, wrx, wrs, wcx, wcs, bp, br, bc):
    n, d = x.shape
    g, sf = sgf.shape
    srf = srt.shape[1]
    return pl.pallas_call(
        _make_tc_pool(n, g),
        out_shape=[
            jax.ShapeDtypeStruct((g, 1), jnp.float32),
            jax.ShapeDtypeStruct((g, 1), jnp.float32),
            jax.ShapeDtypeStruct((g, 1), jnp.float32),
        ],
    )(x, gidf, sgf, srt, wpx, wps, wrx, wrs, wcx, wcs, bp, br, bc)


def kernel(node_embedding, edge_embedding, static_edge_features,
           static_graph_features, static_rt_graph_features, edge_index,
           graph_ids, W_root_0, W_nbr_0, b_0, W_root_1, W_nbr_1, b_1,
           W_root_2, W_nbr_2, b_2, We_o, be_o, Wp_o, bp_o, Wr_o, br_o,
           Wc_o, bc_o):
    n, d = node_embedding.shape
    e = edge_embedding.shape[0]
    g, sf = static_graph_features.shape

    agg_fn, npad, _ = _make_sc_agg(n, d, e)
    edge_fn = _make_sc_edge(n, e)

    src = edge_index[0]
    dst = edge_index[1]
    src2 = src.reshape(e // CHUNK, CHUNK)
    urows = AROWS - 128
    dst3 = jnp.stack([
        jnp.where((dst >= p * urows) & (dst < (p + 1) * urows),
                  dst - p * urows, urows + (dst & 127))
        for p in range(NPH)
    ]).reshape(NPH, e // CHUNK, CHUNK)
    zpt = AROWS // NS
    z2 = jnp.zeros((zpt, d), jnp.float32)
    z1 = jnp.zeros((zpt,), jnp.float32)
    one1 = jnp.ones((CHUNK,), jnp.float32)

    wr_all = jnp.stack([W_root_0, W_root_1, W_root_2])
    wn_all = jnp.stack([W_nbr_0, W_nbr_1, W_nbr_2])
    b_all = jnp.stack([b_0, b_1, b_2]).reshape(3, 1, d)
    wsd = jnp.concatenate([We_o[0:d], We_o[d:2 * d]], axis=1)

    def layer_step(x, ws):
        wr, wn, b = ws
        p, degp = agg_fn(x, src2, dst3, z2, z1, one1)
        x2, y = _tc_layer(x, p, degp.reshape(npad, 1), wr, wn, b, wsd)
        return x2, y

    x, ys = lax.scan(layer_step, node_embedding, (wr_all, wn_all, b_all))
    y = ys[-1]

    contrib = _tc_contrib(edge_embedding, static_edge_features,
                          We_o[2 * d:3 * d], We_o[3 * d:], be_o.reshape(1, 2))
    edge_logits = edge_fn(y.reshape(-1), edge_index[0], edge_index[1],
                          contrib.reshape(-1))

    gidf = graph_ids.reshape(1, n)
    prec, rt, ccs = _tc_pool(
        x, gidf, static_graph_features, static_rt_graph_features,
        Wp_o[:d], Wp_o[d:], Wr_o[:d], Wr_o[d:], Wc_o[:d], Wc_o[d:],
        bp_o.reshape(1, 1), br_o.reshape(1, 1), bc_o.reshape(1, 1))

    return jnp.concatenate([edge_logits, prec.reshape(-1),
                            rt.reshape(-1), ccs.reshape(-1)])


# speedup vs baseline: 2.6853x; 1.5117x over previous
import functools

import jax
import jax.numpy as jnp
from jax import lax
from jax.experimental import pallas as pl
from jax.experimental.pallas import tpu as pltpu
from jax.experimental.pallas import tpu_sc as plsc

NC = 2
NS = 16
NW = NC * NS
CHUNK = 125


NPH = 3
AROWS = 3456


def _make_sc_agg(n, d, e):
    arows = AROWS
    urows = arows
    npad = NPH * urows
    zpt = arows // NS
    upt = urows // NS
    ept = e // NS
    nit = ept // CHUNK
    mesh = plsc.VectorSubcoreMesh(core_axis_name="c", subcore_axis_name="s",
                                  num_cores=1)

    @functools.partial(
        pl.kernel,
        mesh=mesh,
        out_type=jax.ShapeDtypeStruct((npad, d), jnp.float32),
        scratch_types=[
            pltpu.VMEM((nit, CHUNK), jnp.int32),
            pltpu.VMEM((nit, CHUNK), jnp.int32),
            pltpu.VMEM((CHUNK, d), jnp.float32),
            pltpu.VMEM((CHUNK, d), jnp.float32),
            pltpu.VMEM((zpt, d), jnp.float32),
            pltpu.VMEM_SHARED((arows, d), jnp.float32),
            pltpu.SemaphoreType.DMA,
            pltpu.SemaphoreType.DMA,
        ],
    )
    def agg(x_h, src3_h, dst3_h, z2_h, outp,
            sidx, didx, rowsa, rowsb, zbuf, acc, sema, semb):
        s = lax.axis_index("s")
        pltpu.sync_copy(z2_h, zbuf)
        for p in range(NPH):
            pltpu.sync_copy(src3_h.at[p, pl.ds(s * nit, nit)], sidx)
            pltpu.sync_copy(dst3_h.at[p, pl.ds(s * nit, nit)], didx)
            pltpu.sync_copy(zbuf, acc.at[pl.ds(s * zpt, zpt)])
            plsc.subcore_barrier()

            pltpu.async_copy(x_h.at[sidx.at[0]], rowsa, sema)

            def step2(jj, carry):
                j0 = 2 * jj
                j1 = 2 * jj + 1
                pltpu.async_copy(x_h.at[sidx.at[j1]], rowsb, semb)
                pltpu.make_async_copy(x_h.at[sidx.at[j0]], rowsa, sema).wait()
                pltpu.sync_copy(rowsa, acc.at[didx.at[j0]], add=True)

                @pl.when(jj < nit // 2 - 1)
                def _():
                    pltpu.async_copy(x_h.at[sidx.at[j0 + 2]], rowsa, sema)

                pltpu.make_async_copy(x_h.at[sidx.at[j1]], rowsb, semb).wait()
                pltpu.sync_copy(rowsb, acc.at[didx.at[j1]], add=True)
                return carry

            lax.fori_loop(0, nit // 2, step2, 0)
            plsc.subcore_barrier()
            pltpu.sync_copy(acc.at[pl.ds(s * upt, upt)], zbuf)
            pltpu.sync_copy(zbuf, outp.at[pl.ds(p * urows + s * upt, upt)])
            plsc.subcore_barrier()
            pltpu.sync_copy(z2_h, zbuf)

    return agg, npad, urows


def _make_sc_deg(n, e):
    rows = 10240
    rpt = rows // NS
    ept = e // NS
    nit = ept // CHUNK
    mesh = plsc.VectorSubcoreMesh(core_axis_name="c", subcore_axis_name="s",
                                  num_cores=1)

    @functools.partial(
        pl.kernel,
        mesh=mesh,
        out_type=jax.ShapeDtypeStruct((rows,), jnp.float32),
        scratch_types=[
            pltpu.VMEM((nit, CHUNK), jnp.int32),
            pltpu.VMEM((CHUNK,), jnp.float32),
            pltpu.VMEM((rpt,), jnp.float32),
            pltpu.VMEM_SHARED((rows,), jnp.float32),
        ],
    )
    def deg(dst_h, z1_h, one_h, outd, didx, onesv, zd, dacc):
        s = lax.axis_index("s")
        pltpu.sync_copy(dst_h.at[pl.ds(s * nit, nit)], didx)
        pltpu.sync_copy(one_h, onesv)
        pltpu.sync_copy(z1_h, zd)
        pltpu.sync_copy(zd, dacc.at[pl.ds(s * rpt, rpt)])
        plsc.subcore_barrier()

        def step(j, carry):
            pltpu.sync_copy(onesv, dacc.at[didx.at[j]], add=True)
            return carry

        lax.fori_loop(0, nit, step, 0)
        plsc.subcore_barrier()
        pltpu.sync_copy(dacc.at[pl.ds(s * rpt, rpt)], zd)
        pltpu.sync_copy(zd, outd.at[pl.ds(s * rpt, rpt)])

    return deg, rows


def _make_sc_edge(n, e):
    epw = e // NW
    nit = epw // 16
    mesh = plsc.VectorSubcoreMesh(core_axis_name="c", subcore_axis_name="s")

    @functools.partial(
        pl.kernel,
        mesh=mesh,
        out_type=jax.ShapeDtypeStruct((2 * e,), jnp.float32),
        compiler_params=pltpu.CompilerParams(needs_layout_passes=False),
        scratch_types=[
            pltpu.VMEM((4 * n,), jnp.float32),
            pltpu.VMEM((epw,), jnp.int32),
            pltpu.VMEM((epw,), jnp.int32),
            pltpu.VMEM((2 * epw,), jnp.float32),
            pltpu.VMEM((2 * epw,), jnp.float32),
        ],
    )
    def edge(y_h, src_h, dst_h, contrib_h, out_h, ytab, sidx, didx, cv, obuf):
        c = lax.axis_index("c")
        s = lax.axis_index("s")
        wid = c * NS + s
        base = wid * epw
        pltpu.sync_copy(y_h, ytab)
        pltpu.sync_copy(src_h.at[pl.ds(base, epw)], sidx)
        pltpu.sync_copy(dst_h.at[pl.ds(base, epw)], didx)
        pltpu.sync_copy(contrib_h.at[pl.ds(2 * base, 2 * epw)], cv)
        iota = lax.broadcasted_iota(jnp.int32, (16,), 0)

        def step(i, carry):
            s16 = sidx[pl.ds(i * 16, 16)]
            d16 = didx[pl.ds(i * 16, 16)]
            r16 = i * 32 + 2 * iota
            v0 = plsc.load_gather(ytab, [s16 * 4])
            v1 = plsc.load_gather(ytab, [s16 * 4 + 1])
            v2 = plsc.load_gather(ytab, [d16 * 4 + 2])
            v3 = plsc.load_gather(ytab, [d16 * 4 + 3])
            g0 = plsc.load_gather(cv, [r16])
            g1 = plsc.load_gather(cv, [r16 + 1])
            plsc.store_scatter(obuf, [r16], v0 + v2 + g0)
            plsc.store_scatter(obuf, [r16 + 1], v1 + v3 + g1)
            return carry

        lax.fori_loop(0, nit, step, 0)
        pltpu.sync_copy(obuf, out_h.at[pl.ds(2 * base, 2 * epw)])

    return edge


def _tc_layer_body(x_ref, p_ref, dt_ref, wr_ref, wn_ref, b_ref, wsd_ref,
                   o_ref, y_ref):
    inv = 1.0 / jnp.maximum(dt_ref[...], 1.0)
    agg = p_ref[...] * inv
    h = (jnp.dot(x_ref[...], wr_ref[...], preferred_element_type=jnp.float32)
         + jnp.dot(agg, wn_ref[...], preferred_element_type=jnp.float32)
         + b_ref[...])
    xo = jnp.where(h > 0, h, jnp.exp(h) - 1.0)
    o_ref[...] = xo
    y_ref[...] = jnp.dot(xo, wsd_ref[...], preferred_element_type=jnp.float32)


def _tc_layer(x, p, degc, wr, wn, b, wsd, bn=2000):
    n, d = x.shape
    grid = (n // bn,)
    return pl.pallas_call(
        _tc_layer_body,
        grid=grid,
        in_specs=[
            pl.BlockSpec((bn, d), lambda i: (i, 0)),
            pl.BlockSpec((bn, d), lambda i: (i, 0)),
            pl.BlockSpec((bn, 1), lambda i: (i, 0)),
            pl.BlockSpec((d, d), lambda i: (0, 0)),
            pl.BlockSpec((d, d), lambda i: (0, 0)),
            pl.BlockSpec((1, d), lambda i: (0, 0)),
            pl.BlockSpec((d, 4), lambda i: (0, 0)),
        ],
        out_specs=[
            pl.BlockSpec((bn, d), lambda i: (i, 0)),
            pl.BlockSpec((bn, 4), lambda i: (i, 0)),
        ],
        out_shape=[
            jax.ShapeDtypeStruct((n, d), jnp.float32),
            jax.ShapeDtypeStruct((n, 4), jnp.float32),
        ],
    )(x, p, degc, wr, wn, b, wsd)


def _tc_contrib_body(ee_ref, se_ref, wee_ref, wse_ref, b_ref, o_ref):
    o_ref[...] = (
        jnp.dot(ee_ref[...], wee_ref[...], preferred_element_type=jnp.float32)
        + jnp.dot(se_ref[...], wse_ref[...], preferred_element_type=jnp.float32)
        + b_ref[...])


def _tc_contrib(ee, se, wee, wse, b2, be=1280):
    e, d = ee.shape
    sf = se.shape[1]
    return pl.pallas_call(
        _tc_contrib_body,
        grid=(e // be,),
        in_specs=[
            pl.BlockSpec((be, d), lambda i: (i, 0)),
            pl.BlockSpec((be, sf), lambda i: (i, 0)),
            pl.BlockSpec((d, 2), lambda i: (0, 0)),
            pl.BlockSpec((sf, 2), lambda i: (0, 0)),
            pl.BlockSpec((1, 2), lambda i: (0, 0)),
        ],
        out_specs=pl.BlockSpec((be, 2), lambda i: (i, 0)),
        out_shape=jax.ShapeDtypeStruct((e, 2), jnp.float32),
    )(ee, se, wee, wse, b2)


def _make_tc_pool(n, g):
    def body(x_ref, gid_ref, sgf_ref, srt_ref, wpx_ref, wps_ref, wrx_ref,
             wrs_ref, wcx_ref, wcs_ref, bp_ref, br_ref, bc_ref,
             prec_ref, rt_ref, ccs_ref):
        gid = gid_ref[...]
        oh = (lax.broadcasted_iota(jnp.int32, (g, n), 0) == gid)
        ohf = oh.astype(jnp.float32)
        accp = jnp.dot(ohf, x_ref[...], preferred_element_type=jnp.float32)
        accn = jnp.sum(ohf, axis=1, keepdims=True)
        pooled = accp / jnp.maximum(accn, 1.0)
        prec_ref[...] = (
            jnp.dot(pooled, wpx_ref[...], preferred_element_type=jnp.float32)
            + jnp.dot(sgf_ref[...], wps_ref[...], preferred_element_type=jnp.float32)
            + bp_ref[...])
        rt_ref[...] = (
            jnp.dot(pooled, wrx_ref[...], preferred_element_type=jnp.float32)
            + jnp.dot(srt_ref[...], wrs_ref[...], preferred_element_type=jnp.float32)
            + br_ref[...])
        ccs_ref[...] = (
            jnp.dot(pooled, wcx_ref[...], preferred_element_type=jnp.float32)
            + jnp.dot(srt_ref[...], wcs_ref[...], preferred_element_type=jnp.float32)
            + bc_ref[...])

    return body


def _tc_pool(x, gidf, sgf, srt, wpx, wps, wrx, wrs, wcx, wcs, bp, br, bc):
    n, d = x.shape
    g, sf = sgf.shape
    srf = srt.shape[1]
    return pl.pallas_call(
        _make_tc_pool(n, g),
        out_shape=[
            jax.ShapeDtypeStruct((g, 1), jnp.float32),
            jax.ShapeDtypeStruct((g, 1), jnp.float32),
            jax.ShapeDtypeStruct((g, 1), jnp.float32),
        ],
    )(x, gidf, sgf, srt, wpx, wps, wrx, wrs, wcx, wcs, bp, br, bc)


def kernel(node_embedding, edge_embedding, static_edge_features,
           static_graph_features, static_rt_graph_features, edge_index,
           graph_ids, W_root_0, W_nbr_0, b_0, W_root_1, W_nbr_1, b_1,
           W_root_2, W_nbr_2, b_2, We_o, be_o, Wp_o, bp_o, Wr_o, br_o,
           Wc_o, bc_o):
    n, d = node_embedding.shape
    e = edge_embedding.shape[0]
    g, sf = static_graph_features.shape

    agg_fn, npad, _ = _make_sc_agg(n, d, e)
    edge_fn = _make_sc_edge(n, e)

    src = edge_index[0]
    dst = edge_index[1]
    urows = AROWS
    inph = [(dst >= p * urows) & (dst < (p + 1) * urows) for p in range(NPH)]
    src3 = jnp.stack([
        jnp.where(inph[p], src, n + (dst & 127)) for p in range(NPH)
    ]).reshape(NPH, e // CHUNK, CHUNK)
    dst3 = jnp.stack([
        jnp.where(inph[p], dst - p * urows, dst % urows) for p in range(NPH)
    ]).reshape(NPH, e // CHUNK, CHUNK)
    zpt = AROWS // NS
    z2 = jnp.zeros((zpt, d), jnp.float32)
    zrows = jnp.zeros((128, d), jnp.float32)

    deg_fn, drows = _make_sc_deg(n, e)
    dst2 = dst.reshape(e // CHUNK, CHUNK)
    z1d = jnp.zeros((drows // NS,), jnp.float32)
    one1 = jnp.ones((CHUNK,), jnp.float32)
    degc = deg_fn(dst2, z1d, one1).reshape(drows, 1)

    wr_all = jnp.stack([W_root_0, W_root_1, W_root_2])
    wn_all = jnp.stack([W_nbr_0, W_nbr_1, W_nbr_2])
    b_all = jnp.stack([b_0, b_1, b_2]).reshape(3, 1, d)
    wsd = jnp.concatenate([We_o[0:d], We_o[d:2 * d]], axis=1)

    def layer_step(x, ws):
        wr, wn, b = ws
        xp = jnp.concatenate([x, zrows])
        p = agg_fn(xp, src3, dst3, z2)
        x2, y = _tc_layer(x, p, degc, wr, wn, b, wsd)
        return x2, y

    x, ys = lax.scan(layer_step, node_embedding, (wr_all, wn_all, b_all))
    y = ys[-1]

    contrib = _tc_contrib(edge_embedding, static_edge_features,
                          We_o[2 * d:3 * d], We_o[3 * d:], be_o.reshape(1, 2))
    edge_logits = edge_fn(y.reshape(-1), edge_index[0], edge_index[1],
                          contrib.reshape(-1))

    gidf = graph_ids.reshape(1, n)
    prec, rt, ccs = _tc_pool(
        x, gidf, static_graph_features, static_rt_graph_features,
        Wp_o[:d], Wp_o[d:], Wr_o[:d], Wr_o[d:], Wc_o[:d], Wc_o[d:],
        bp_o.reshape(1, 1), br_o.reshape(1, 1), bc_o.reshape(1, 1))

    return jnp.concatenate([edge_logits, prec.reshape(-1),
                            rt.reshape(-1), ccs.reshape(-1)])


# speedup vs baseline: 2.6902x; 1.0018x over previous
import functools

import jax
import jax.numpy as jnp
from jax import lax
from jax.experimental import pallas as pl
from jax.experimental.pallas import tpu as pltpu
from jax.experimental.pallas import tpu_sc as plsc

NC = 2
NS = 16
NW = NC * NS
CHUNK = 125


NPH = 3
AROWS = 3456


def _make_sc_agg(n, d, e):
    arows = AROWS
    urows = arows
    npad = NPH * urows
    zpt = arows // NS
    upt = urows // NS
    ept = e // NS
    nit = ept // CHUNK
    mesh = plsc.VectorSubcoreMesh(core_axis_name="c", subcore_axis_name="s",
                                  num_cores=1)

    @functools.partial(
        pl.kernel,
        mesh=mesh,
        out_type=jax.ShapeDtypeStruct((npad, d), jnp.float32),
        scratch_types=[
            pltpu.VMEM((nit, CHUNK), jnp.int32),
            pltpu.VMEM((nit, CHUNK), jnp.int32),
            pltpu.VMEM((CHUNK, d), jnp.float32),
            pltpu.VMEM((CHUNK, d), jnp.float32),
            pltpu.VMEM((zpt, d), jnp.float32),
            pltpu.VMEM_SHARED((arows, d), jnp.float32),
            pltpu.SemaphoreType.DMA,
            pltpu.SemaphoreType.DMA,
            pltpu.SemaphoreType.DMA,
            pltpu.SemaphoreType.DMA,
        ],
    )
    def agg(x_h, src3_h, dst3_h, z2_h, outp,
            sidx, didx, rowsa, rowsb, zbuf, acc, sema, semb, semsa, semsb):
        s = lax.axis_index("s")
        pltpu.sync_copy(z2_h, zbuf)
        for p in range(NPH):
            pltpu.sync_copy(src3_h.at[p, pl.ds(s * nit, nit)], sidx)
            pltpu.sync_copy(dst3_h.at[p, pl.ds(s * nit, nit)], didx)
            pltpu.sync_copy(zbuf, acc.at[pl.ds(s * zpt, zpt)])
            plsc.subcore_barrier()

            pltpu.async_copy(x_h.at[sidx.at[0]], rowsa, sema)

            def step2(jj, carry):
                j0 = 2 * jj
                j1 = 2 * jj + 1

                @pl.when(jj > 0)
                def _():
                    pltpu.make_async_copy(rowsb, acc.at[didx.at[j1 - 2]],
                                          semsb).wait()

                pltpu.async_copy(x_h.at[sidx.at[j1]], rowsb, semb)
                pltpu.make_async_copy(x_h.at[sidx.at[j0]], rowsa, sema).wait()
                pltpu.async_copy(rowsa, acc.at[didx.at[j0]], semsa, add=True)
                pltpu.make_async_copy(rowsa, acc.at[didx.at[j0]], semsa).wait()

                @pl.when(jj < nit // 2 - 1)
                def _():
                    pltpu.async_copy(x_h.at[sidx.at[j0 + 2]], rowsa, sema)

                pltpu.make_async_copy(x_h.at[sidx.at[j1]], rowsb, semb).wait()
                pltpu.async_copy(rowsb, acc.at[didx.at[j1]], semsb, add=True)
                return carry

            lax.fori_loop(0, nit // 2, step2, 0)
            pltpu.make_async_copy(rowsb, acc.at[didx.at[nit - 1]],
                                  semsb).wait()
            plsc.subcore_barrier()
            pltpu.sync_copy(acc.at[pl.ds(s * upt, upt)], zbuf)
            pltpu.sync_copy(zbuf, outp.at[pl.ds(p * urows + s * upt, upt)])
            plsc.subcore_barrier()
            pltpu.sync_copy(z2_h, zbuf)

    return agg, npad, urows


def _make_sc_deg(n, e):
    rows = 10240
    rpt = rows // NS
    ept = e // NS
    nit = ept // CHUNK
    mesh = plsc.VectorSubcoreMesh(core_axis_name="c", subcore_axis_name="s",
                                  num_cores=1)

    @functools.partial(
        pl.kernel,
        mesh=mesh,
        out_type=jax.ShapeDtypeStruct((rows,), jnp.float32),
        scratch_types=[
            pltpu.VMEM((nit, CHUNK), jnp.int32),
            pltpu.VMEM((CHUNK,), jnp.float32),
            pltpu.VMEM((rpt,), jnp.float32),
            pltpu.VMEM_SHARED((rows,), jnp.float32),
        ],
    )
    def deg(dst_h, z1_h, one_h, outd, didx, onesv, zd, dacc):
        s = lax.axis_index("s")
        pltpu.sync_copy(dst_h.at[pl.ds(s * nit, nit)], didx)
        pltpu.sync_copy(one_h, onesv)
        pltpu.sync_copy(z1_h, zd)
        pltpu.sync_copy(zd, dacc.at[pl.ds(s * rpt, rpt)])
        plsc.subcore_barrier()

        def step(j, carry):
            pltpu.sync_copy(onesv, dacc.at[didx.at[j]], add=True)
            return carry

        lax.fori_loop(0, nit, step, 0)
        plsc.subcore_barrier()
        pltpu.sync_copy(dacc.at[pl.ds(s * rpt, rpt)], zd)
        pltpu.sync_copy(zd, outd.at[pl.ds(s * rpt, rpt)])

    return deg, rows


def _make_sc_edge(n, e):
    epw = e // NW
    nit = epw // 16
    mesh = plsc.VectorSubcoreMesh(core_axis_name="c", subcore_axis_name="s")

    @functools.partial(
        pl.kernel,
        mesh=mesh,
        out_type=jax.ShapeDtypeStruct((2 * e,), jnp.float32),
        compiler_params=pltpu.CompilerParams(needs_layout_passes=False),
        scratch_types=[
            pltpu.VMEM((4 * n,), jnp.float32),
            pltpu.VMEM((epw,), jnp.int32),
            pltpu.VMEM((epw,), jnp.int32),
            pltpu.VMEM((2 * epw,), jnp.float32),
            pltpu.VMEM((2 * epw,), jnp.float32),
        ],
    )
    def edge(y_h, src_h, dst_h, contrib_h, out_h, ytab, sidx, didx, cv, obuf):
        c = lax.axis_index("c")
        s = lax.axis_index("s")
        wid = c * NS + s
        base = wid * epw
        pltpu.sync_copy(y_h, ytab)
        pltpu.sync_copy(src_h.at[pl.ds(base, epw)], sidx)
        pltpu.sync_copy(dst_h.at[pl.ds(base, epw)], didx)
        pltpu.sync_copy(contrib_h.at[pl.ds(2 * base, 2 * epw)], cv)
        iota = lax.broadcasted_iota(jnp.int32, (16,), 0)

        def step(i, carry):
            s16 = sidx[pl.ds(i * 16, 16)]
            d16 = didx[pl.ds(i * 16, 16)]
            r16 = i * 32 + 2 * iota
            v0 = plsc.load_gather(ytab, [s16 * 4])
            v1 = plsc.load_gather(ytab, [s16 * 4 + 1])
            v2 = plsc.load_gather(ytab, [d16 * 4 + 2])
            v3 = plsc.load_gather(ytab, [d16 * 4 + 3])
            g0 = plsc.load_gather(cv, [r16])
            g1 = plsc.load_gather(cv, [r16 + 1])
            plsc.store_scatter(obuf, [r16], v0 + v2 + g0)
            plsc.store_scatter(obuf, [r16 + 1], v1 + v3 + g1)
            return carry

        lax.fori_loop(0, nit, step, 0)
        pltpu.sync_copy(obuf, out_h.at[pl.ds(2 * base, 2 * epw)])

    return edge


def _tc_layer_body(x_ref, p_ref, dt_ref, wr_ref, wn_ref, b_ref, wsd_ref,
                   o_ref, y_ref):
    inv = 1.0 / jnp.maximum(dt_ref[...], 1.0)
    agg = p_ref[...] * inv
    h = (jnp.dot(x_ref[...], wr_ref[...], preferred_element_type=jnp.float32)
         + jnp.dot(agg, wn_ref[...], preferred_element_type=jnp.float32)
         + b_ref[...])
    xo = jnp.where(h > 0, h, jnp.exp(h) - 1.0)
    o_ref[...] = xo
    y_ref[...] = jnp.dot(xo, wsd_ref[...], preferred_element_type=jnp.float32)


def _tc_layer(x, p, degc, wr, wn, b, wsd, bn=2000):
    n, d = x.shape
    grid = (n // bn,)
    return pl.pallas_call(
        _tc_layer_body,
        grid=grid,
        in_specs=[
            pl.BlockSpec((bn, d), lambda i: (i, 0)),
            pl.BlockSpec((bn, d), lambda i: (i, 0)),
            pl.BlockSpec((bn, 1), lambda i: (i, 0)),
            pl.BlockSpec((d, d), lambda i: (0, 0)),
            pl.BlockSpec((d, d), lambda i: (0, 0)),
            pl.BlockSpec((1, d), lambda i: (0, 0)),
            pl.BlockSpec((d, 4), lambda i: (0, 0)),
        ],
        out_specs=[
            pl.BlockSpec((bn, d), lambda i: (i, 0)),
            pl.BlockSpec((bn, 4), lambda i: (i, 0)),
        ],
        out_shape=[
            jax.ShapeDtypeStruct((n, d), jnp.float32),
            jax.ShapeDtypeStruct((n, 4), jnp.float32),
        ],
    )(x, p, degc, wr, wn, b, wsd)


def _tc_contrib_body(ee_ref, se_ref, wee_ref, wse_ref, b_ref, o_ref):
    o_ref[...] = (
        jnp.dot(ee_ref[...], wee_ref[...], preferred_element_type=jnp.float32)
        + jnp.dot(se_ref[...], wse_ref[...], preferred_element_type=jnp.float32)
        + b_ref[...])


def _tc_contrib(ee, se, wee, wse, b2, be=1280):
    e, d = ee.shape
    sf = se.shape[1]
    return pl.pallas_call(
        _tc_contrib_body,
        grid=(e // be,),
        in_specs=[
            pl.BlockSpec((be, d), lambda i: (i, 0)),
            pl.BlockSpec((be, sf), lambda i: (i, 0)),
            pl.BlockSpec((d, 2), lambda i: (0, 0)),
            pl.BlockSpec((sf, 2), lambda i: (0, 0)),
            pl.BlockSpec((1, 2), lambda i: (0, 0)),
        ],
        out_specs=pl.BlockSpec((be, 2), lambda i: (i, 0)),
        out_shape=jax.ShapeDtypeStruct((e, 2), jnp.float32),
    )(ee, se, wee, wse, b2)


def _make_tc_pool(n, g):
    def body(x_ref, gid_ref, sgf_ref, srt_ref, wpx_ref, wps_ref, wrx_ref,
             wrs_ref, wcx_ref, wcs_ref, bp_ref, br_ref, bc_ref,
             prec_ref, rt_ref, ccs_ref):
        gid = gid_ref[...]
        oh = (lax.broadcasted_iota(jnp.int32, (g, n), 0) == gid)
        ohf = oh.astype(jnp.float32)
        accp = jnp.dot(ohf, x_ref[...], preferred_element_type=jnp.float32)
        accn = jnp.sum(ohf, axis=1, keepdims=True)
        pooled = accp / jnp.maximum(accn, 1.0)
        prec_ref[...] = (
            jnp.dot(pooled, wpx_ref[...], preferred_element_type=jnp.float32)
            + jnp.dot(sgf_ref[...], wps_ref[...], preferred_element_type=jnp.float32)
            + bp_ref[...])
        rt_ref[...] = (
            jnp.dot(pooled, wrx_ref[...], preferred_element_type=jnp.float32)
            + jnp.dot(srt_ref[...], wrs_ref[...], preferred_element_type=jnp.float32)
            + br_ref[...])
        ccs_ref[...] = (
            jnp.dot(pooled, wcx_ref[...], preferred_element_type=jnp.float32)
            + jnp.dot(srt_ref[...], wcs_ref[...], preferred_element_type=jnp.float32)
            + bc_ref[...])

    return body


def _tc_pool(x, gidf, sgf, srt, wpx, wps, wrx, wrs, wcx, wcs, bp, br, bc):
    n, d = x.shape
    g, sf = sgf.shape
    srf = srt.shape[1]
    return pl.pallas_call(
        _make_tc_pool(n, g),
        out_shape=[
            jax.ShapeDtypeStruct((g, 1), jnp.float32),
            jax.ShapeDtypeStruct((g, 1), jnp.float32),
            jax.ShapeDtypeStruct((g, 1), jnp.float32),
        ],
    )(x, gidf, sgf, srt, wpx, wps, wrx, wrs, wcx, wcs, bp, br, bc)


def kernel(node_embedding, edge_embedding, static_edge_features,
           static_graph_features, static_rt_graph_features, edge_index,
           graph_ids, W_root_0, W_nbr_0, b_0, W_root_1, W_nbr_1, b_1,
           W_root_2, W_nbr_2, b_2, We_o, be_o, Wp_o, bp_o, Wr_o, br_o,
           Wc_o, bc_o):
    n, d = node_embedding.shape
    e = edge_embedding.shape[0]
    g, sf = static_graph_features.shape

    agg_fn, npad, _ = _make_sc_agg(n, d, e)
    edge_fn = _make_sc_edge(n, e)

    src = edge_index[0]
    dst = edge_index[1]
    urows = AROWS
    inph = [(dst >= p * urows) & (dst < (p + 1) * urows) for p in range(NPH)]
    src3 = jnp.stack([
        jnp.where(inph[p], src, n + (dst & 127)) for p in range(NPH)
    ]).reshape(NPH, e // CHUNK, CHUNK)
    dst3 = jnp.stack([
        jnp.where(inph[p], dst - p * urows, dst % urows) for p in range(NPH)
    ]).reshape(NPH, e // CHUNK, CHUNK)
    zpt = AROWS // NS
    z2 = jnp.zeros((zpt, d), jnp.float32)
    zrows = jnp.zeros((128, d), jnp.float32)

    deg_fn, drows = _make_sc_deg(n, e)
    dst2 = dst.reshape(e // CHUNK, CHUNK)
    z1d = jnp.zeros((drows // NS,), jnp.float32)
    one1 = jnp.ones((CHUNK,), jnp.float32)
    degc = deg_fn(dst2, z1d, one1).reshape(drows, 1)

    wr_all = jnp.stack([W_root_0, W_root_1, W_root_2])
    wn_all = jnp.stack([W_nbr_0, W_nbr_1, W_nbr_2])
    b_all = jnp.stack([b_0, b_1, b_2]).reshape(3, 1, d)
    wsd = jnp.concatenate([We_o[0:d], We_o[d:2 * d]], axis=1)

    def layer_step(x, ws):
        wr, wn, b = ws
        xp = jnp.concatenate([x, zrows])
        p = agg_fn(xp, src3, dst3, z2)
        x2, y = _tc_layer(x, p, degc, wr, wn, b, wsd)
        return x2, y

    x, ys = lax.scan(layer_step, node_embedding, (wr_all, wn_all, b_all))
    y = ys[-1]

    contrib = _tc_contrib(edge_embedding, static_edge_features,
                          We_o[2 * d:3 * d], We_o[3 * d:], be_o.reshape(1, 2))
    edge_logits = edge_fn(y.reshape(-1), edge_index[0], edge_index[1],
                          contrib.reshape(-1))

    gidf = graph_ids.reshape(1, n)
    prec, rt, ccs = _tc_pool(
        x, gidf, static_graph_features, static_rt_graph_features,
        Wp_o[:d], Wp_o[d:], Wr_o[:d], Wr_o[d:], Wc_o[:d], Wc_o[d:],
        bp_o.reshape(1, 1), br_o.reshape(1, 1), bc_o.reshape(1, 1))

    return jnp.concatenate([edge_logits, prec.reshape(-1),
                            rt.reshape(-1), ccs.reshape(-1)])
